# Initial kernel scaffold; baseline (speedup 1.0000x reference)
import jax
import jax.numpy as jnp
from jax.experimental import pallas as pl


def kernel(x, edge_index, batch, W1, b1, W2, b2):
    raise NotImplementedError("write your pallas kernel here")


# speedup vs baseline: 8.8471x; 8.8471x over previous
import functools

import jax
import jax.numpy as jnp
from jax import lax
from jax.experimental import pallas as pl
from jax.experimental.pallas import tpu as pltpu
from jax.experimental.pallas import tpu_sc as plsc

N = 10000
E = 320000
DIN = 128
DH = 128
DOUT = 64
G = 128

NC = 2
NS = 16
NW = NC * NS
LB = 128
KB = 80
EP = NW * KB * LB
NP = 10112
STRIPE = NP // NS

RB = 1000
NB = N // RB


def _sc_deg(dstw, z1, ones):
    mesh = plsc.VectorSubcoreMesh(core_axis_name="c", subcore_axis_name="s")

    @functools.partial(
        pl.kernel,
        out_type=jax.ShapeDtypeStruct((NC * NP,), jnp.float32),
        mesh=mesh,
        scratch_types=[
            pltpu.VMEM((KB, LB), jnp.int32),
            pltpu.VMEM((LB,), jnp.float32),
            pltpu.VMEM((STRIPE,), jnp.float32),
            pltpu.VMEM_SHARED((NP,), jnp.float32),
            pltpu.SemaphoreType.DMA,
        ],
    )
    def k(dstw_h, z_h, ones_h, out_h, idx_v, ones_v, stripe_v, acc, sem):
        c = lax.axis_index("c")
        s = lax.axis_index("s")
        wid = s * NC + c
        pltpu.sync_copy(z_h.at[pl.ds(s * STRIPE, STRIPE)], stripe_v)
        pltpu.sync_copy(stripe_v, acc.at[pl.ds(s * STRIPE, STRIPE)])
        pltpu.sync_copy(dstw_h.at[wid], idx_v)
        pltpu.sync_copy(ones_h, ones_v)
        plsc.subcore_barrier()

        def body(j, carry):
            pltpu.sync_copy(ones_v, acc.at[idx_v.at[j]], add=True)
            return carry

        lax.fori_loop(0, KB, body, 0)
        plsc.subcore_barrier()
        pltpu.sync_copy(acc.at[pl.ds(s * STRIPE, STRIPE)], stripe_v)
        pltpu.sync_copy(stripe_v, out_h.at[pl.ds(c * NP + s * STRIPE, STRIPE)])

    return k(dstw, z1, ones)


def _sc_scatter(table, srcw, dstw, zd, d):
    mesh = plsc.VectorSubcoreMesh(core_axis_name="c", subcore_axis_name="s")

    @functools.partial(
        pl.kernel,
        out_type=jax.ShapeDtypeStruct((NC, NP, d), jnp.float32),
        mesh=mesh,
        scratch_types=[
            pltpu.VMEM((KB, LB), jnp.int32),
            pltpu.VMEM((KB, LB), jnp.int32),
            pltpu.VMEM((LB, d), jnp.float32),
            pltpu.VMEM_SHARED((NP, d), jnp.float32),
            pltpu.SemaphoreType.DMA,
        ],
    )
    def k(table_h, srcw_h, dstw_h, z_h, out_h, idx_s, idx_d, rows_v, acc, sem):
        c = lax.axis_index("c")
        s = lax.axis_index("s")
        wid = s * NC + c
        pltpu.sync_copy(z_h.at[pl.ds(s * STRIPE, STRIPE)],
                        acc.at[pl.ds(s * STRIPE, STRIPE)])
        pltpu.sync_copy(srcw_h.at[wid], idx_s)
        pltpu.sync_copy(dstw_h.at[wid], idx_d)
        plsc.subcore_barrier()

        def body(j, carry):
            pltpu.async_copy(table_h.at[idx_s.at[j]], rows_v, sem).wait()
            pltpu.sync_copy(rows_v, acc.at[idx_d.at[j]], add=True)
            return carry

        lax.fori_loop(0, KB, body, 0)
        plsc.subcore_barrier()
        pltpu.sync_copy(acc.at[pl.ds(s * STRIPE, STRIPE)],
                        out_h.at[c, pl.ds(s * STRIPE, STRIPE)])

    return k(table, srcw, dstw, zd)


def _tc_lin1(x, W1, degt):
    def body(deg_ref, x_ref, w_ref, out_ref):
        dinv = lax.rsqrt(deg_ref[:, 0] + deg_ref[:, 1] + 1.0)
        hw = jnp.dot(x_ref[...], w_ref[...], preferred_element_type=jnp.float32)
        out_ref[...] = hw * dinv[:, None]

    return pl.pallas_call(
        body,
        grid=(NB,),
        in_specs=[
            pl.BlockSpec((RB, NC), lambda i: (i, 0)),
            pl.BlockSpec((RB, DIN), lambda i: (i, 0)),
            pl.BlockSpec((DIN, DH), lambda i: (0, 0)),
        ],
        out_specs=pl.BlockSpec((RB, DH), lambda i: (i, 0)),
        out_shape=jax.ShapeDtypeStruct((NP, DH), jnp.float32),
    )(degt, x, W1)


def _tc_lin2(s1p, hw1p, degt, b1, W2):
    def body(deg_ref, s_ref, hw_ref, b_ref, w_ref, out_ref):
        dinv = lax.rsqrt(deg_ref[:, 0] + deg_ref[:, 1] + 1.0)
        tot = s_ref[0] + s_ref[1] + hw_ref[...]
        h = jnp.maximum(tot * dinv[:, None] + b_ref[...][None, :], 0.0)
        hw2 = jnp.dot(h, w_ref[...],
                      preferred_element_type=jnp.float32) * dinv[:, None]
        out_ref[...] = jnp.concatenate([hw2, jnp.zeros_like(hw2)], axis=1)

    return pl.pallas_call(
        body,
        grid=(NB,),
        in_specs=[
            pl.BlockSpec((RB, NC), lambda i: (i, 0)),
            pl.BlockSpec((NC, RB, DH), lambda i: (0, i, 0)),
            pl.BlockSpec((RB, DH), lambda i: (i, 0)),
            pl.BlockSpec((DH,), lambda i: (0,)),
            pl.BlockSpec((DH, DOUT), lambda i: (0, 0)),
        ],
        out_specs=pl.BlockSpec((RB, DH), lambda i: (i, 0)),
        out_shape=jax.ShapeDtypeStruct((NP, DH), jnp.float32),
    )(degt, s1p, hw1p, b1, W2)


def _tc_out(s2p, hw2p, degt, b2, batch3):
    def body(deg_ref, s_ref, hw_ref, b_ref, batch_ref, h2_ref, hs_ref):
        i = pl.program_id(0)
        dinv = lax.rsqrt(deg_ref[:, 0] + deg_ref[:, 1] + 1.0)
        tot = s_ref[0, :, :DOUT] + s_ref[1, :, :DOUT] + hw_ref[:, :DOUT]
        h2 = tot * dinv[:, None] + b_ref[...][None, :]
        h2_ref[...] = h2
        bvec = jnp.broadcast_to(batch_ref[0], (G, RB))
        gids = lax.broadcasted_iota(jnp.int32, (G, RB), 0)
        onehot = (bvec == gids).astype(jnp.float32)
        contrib = jnp.dot(onehot, h2, preferred_element_type=jnp.float32)

        @pl.when(i == 0)
        def _():
            hs_ref[...] = contrib

        @pl.when(i > 0)
        def _():
            hs_ref[...] += contrib

    return pl.pallas_call(
        body,
        grid=(NB,),
        in_specs=[
            pl.BlockSpec((RB, NC), lambda i: (i, 0)),
            pl.BlockSpec((NC, RB, DH), lambda i: (0, i, 0)),
            pl.BlockSpec((RB, DH), lambda i: (i, 0)),
            pl.BlockSpec((DOUT,), lambda i: (0,)),
            pl.BlockSpec((1, 1, RB), lambda i: (i, 0, 0)),
        ],
        out_specs=[
            pl.BlockSpec((RB, DOUT), lambda i: (i, 0)),
            pl.BlockSpec((G, DOUT), lambda i: (0, 0)),
        ],
        out_shape=[
            jax.ShapeDtypeStruct((N, DOUT), jnp.float32),
            jax.ShapeDtypeStruct((G, DOUT), jnp.float32),
        ],
    )(degt, s2p, hw2p, b2, batch3)


def kernel(x, edge_index, batch, W1, b1, W2, b2):
    src = edge_index[1]
    dst = edge_index[0]
    pad = jnp.full((EP - E,), N, dtype=jnp.int32)
    srcw = jnp.concatenate([src, pad]).reshape(NW, KB, LB)
    dstw = jnp.concatenate([dst, pad]).reshape(NW, KB, LB)

    z1 = jnp.zeros((NP,), jnp.float32)
    z128 = jnp.zeros((NP, DH), jnp.float32)
    ones = jnp.ones((LB,), jnp.float32)

    degp = _sc_deg(dstw, z1, ones).reshape(NC, NP)
    degt = degp.T
    hw1p = _tc_lin1(x, W1, degt)
    s1p = _sc_scatter(hw1p, srcw, dstw, z128, DH)
    hw2p = _tc_lin2(s1p, hw1p, degt, b1, W2)
    s2p = _sc_scatter(hw2p, srcw, dstw, z128, DH)
    batch3 = batch.reshape(NB, 1, RB)
    h2, hs = _tc_out(s2p, hw2p, degt, b2, batch3)
    return (hs, h2)


# speedup vs baseline: 9.9649x; 1.1263x over previous
import functools

import jax
import jax.numpy as jnp
from jax import lax
from jax.experimental import pallas as pl
from jax.experimental.pallas import tpu as pltpu
from jax.experimental.pallas import tpu_sc as plsc

N = 10000
E = 320000
DIN = 128
DH = 128
DOUT = 64
G = 128

NC = 2
NS = 16
NW = NC * NS
LB = 128
KB = 80
CH = 40
EP = NW * KB * LB
NP = 10112
STRIPE = NP // NS

RB = 1000
NB = N // RB


def _sc_deg(dstw, z1, ones):
    mesh = plsc.VectorSubcoreMesh(core_axis_name="c", subcore_axis_name="s")

    @functools.partial(
        pl.kernel,
        out_type=jax.ShapeDtypeStruct((NC * NP,), jnp.float32),
        mesh=mesh,
        scratch_types=[
            pltpu.VMEM((KB, LB), jnp.int32),
            pltpu.VMEM((LB,), jnp.float32),
            pltpu.VMEM((STRIPE,), jnp.float32),
            pltpu.VMEM_SHARED((NP,), jnp.float32),
            pltpu.SemaphoreType.DMA,
        ],
    )
    def k(dstw_h, z_h, ones_h, out_h, idx_v, ones_v, stripe_v, acc, sem):
        c = lax.axis_index("c")
        s = lax.axis_index("s")
        wid = s * NC + c
        pltpu.sync_copy(z_h.at[pl.ds(s * STRIPE, STRIPE)], stripe_v)
        pltpu.sync_copy(stripe_v, acc.at[pl.ds(s * STRIPE, STRIPE)])
        pltpu.sync_copy(dstw_h.at[wid], idx_v)
        pltpu.sync_copy(ones_h, ones_v)
        plsc.subcore_barrier()

        def body(j, carry):
            pltpu.sync_copy(ones_v, acc.at[idx_v.at[j]], add=True)
            return carry

        lax.fori_loop(0, KB, body, 0)
        plsc.subcore_barrier()
        pltpu.sync_copy(acc.at[pl.ds(s * STRIPE, STRIPE)], stripe_v)
        pltpu.sync_copy(stripe_v, out_h.at[pl.ds(c * NP + s * STRIPE, STRIPE)])

    return k(dstw, z1, ones)


def _sc_scatter(table, srcw, dstw, zd, d):
    mesh = plsc.VectorSubcoreMesh(core_axis_name="c", subcore_axis_name="s")

    @functools.partial(
        pl.kernel,
        out_type=jax.ShapeDtypeStruct((NC, NP, d), jnp.float32),
        mesh=mesh,
        scratch_types=[
            pltpu.VMEM((CH, LB), jnp.int32),
            pltpu.VMEM((CH, LB), jnp.int32),
            pltpu.VMEM((LB, d), jnp.float32),
            pltpu.VMEM((LB, d), jnp.float32),
            pltpu.VMEM_SHARED((NP, d), jnp.float32),
            pltpu.SemaphoreType.DMA,
            pltpu.SemaphoreType.DMA,
        ],
    )
    def k(table_h, srcw_h, dstw_h, z_h, out_h, idx_s, idx_d,
          r0, r1, acc, s0, s1):
        rows = (r0, r1)
        sems = (s0, s1)
        nbuf = len(rows)
        c = lax.axis_index("c")
        s = lax.axis_index("s")
        wid = s * NC + c
        pltpu.sync_copy(z_h.at[pl.ds(s * STRIPE, STRIPE)],
                        acc.at[pl.ds(s * STRIPE, STRIPE)])
        plsc.subcore_barrier()

        for q in range(KB // CH):
            pltpu.sync_copy(srcw_h.at[wid, pl.ds(q * CH, CH)], idx_s)
            pltpu.sync_copy(dstw_h.at[wid, pl.ds(q * CH, CH)], idx_d)
            for b in range(nbuf):
                pltpu.async_copy(table_h.at[idx_s.at[b]], rows[b], sems[b])

            def body(g, carry):
                for b in range(nbuf):
                    j = g * nbuf + b
                    pltpu.make_async_copy(table_h.at[idx_s.at[j]],
                                          rows[b], sems[b]).wait()
                    pltpu.sync_copy(rows[b], acc.at[idx_d.at[j]], add=True)
                    jn = j + nbuf

                    @pl.when(jn < CH)
                    def _():
                        pltpu.async_copy(table_h.at[idx_s.at[jn]],
                                         rows[b], sems[b])
                return carry

            lax.fori_loop(0, CH // nbuf, body, 0)
        plsc.subcore_barrier()
        pltpu.sync_copy(acc.at[pl.ds(s * STRIPE, STRIPE)],
                        out_h.at[c, pl.ds(s * STRIPE, STRIPE)])

    return k(table, srcw, dstw, zd)


def _tc_lin1(x, W1, degt):
    def body(deg_ref, x_ref, w_ref, out_ref):
        dinv = lax.rsqrt(deg_ref[:, 0] + deg_ref[:, 1] + 1.0)
        hw = jnp.dot(x_ref[...], w_ref[...], preferred_element_type=jnp.float32)
        out_ref[...] = hw * dinv[:, None]

    return pl.pallas_call(
        body,
        grid=(NB,),
        in_specs=[
            pl.BlockSpec((RB, NC), lambda i: (i, 0)),
            pl.BlockSpec((RB, DIN), lambda i: (i, 0)),
            pl.BlockSpec((DIN, DH), lambda i: (0, 0)),
        ],
        out_specs=pl.BlockSpec((RB, DH), lambda i: (i, 0)),
        out_shape=jax.ShapeDtypeStruct((NP, DH), jnp.float32),
    )(degt, x, W1)


def _tc_lin2(s1p, hw1p, degt, b1, W2):
    def body(deg_ref, s_ref, hw_ref, b_ref, w_ref, out_ref):
        dinv = lax.rsqrt(deg_ref[:, 0] + deg_ref[:, 1] + 1.0)
        tot = s_ref[0] + s_ref[1] + hw_ref[...]
        h = jnp.maximum(tot * dinv[:, None] + b_ref[...][None, :], 0.0)
        hw2 = jnp.dot(h, w_ref[...],
                      preferred_element_type=jnp.float32) * dinv[:, None]
        out_ref[...] = jnp.concatenate([hw2, jnp.zeros_like(hw2)], axis=1)

    return pl.pallas_call(
        body,
        grid=(NB,),
        in_specs=[
            pl.BlockSpec((RB, NC), lambda i: (i, 0)),
            pl.BlockSpec((NC, RB, DH), lambda i: (0, i, 0)),
            pl.BlockSpec((RB, DH), lambda i: (i, 0)),
            pl.BlockSpec((DH,), lambda i: (0,)),
            pl.BlockSpec((DH, DOUT), lambda i: (0, 0)),
        ],
        out_specs=pl.BlockSpec((RB, DH), lambda i: (i, 0)),
        out_shape=jax.ShapeDtypeStruct((NP, DH), jnp.float32),
    )(degt, s1p, hw1p, b1, W2)


def _tc_out(s2p, hw2p, degt, b2, batch3):
    def body(deg_ref, s_ref, hw_ref, b_ref, batch_ref, h2_ref, hs_ref):
        i = pl.program_id(0)
        dinv = lax.rsqrt(deg_ref[:, 0] + deg_ref[:, 1] + 1.0)
        tot = s_ref[0, :, :DOUT] + s_ref[1, :, :DOUT] + hw_ref[:, :DOUT]
        h2 = tot * dinv[:, None] + b_ref[...][None, :]
        h2_ref[...] = h2
        bvec = jnp.broadcast_to(batch_ref[0], (G, RB))
        gids = lax.broadcasted_iota(jnp.int32, (G, RB), 0)
        onehot = (bvec == gids).astype(jnp.float32)
        contrib = jnp.dot(onehot, h2, preferred_element_type=jnp.float32)

        @pl.when(i == 0)
        def _():
            hs_ref[...] = contrib

        @pl.when(i > 0)
        def _():
            hs_ref[...] += contrib

    return pl.pallas_call(
        body,
        grid=(NB,),
        in_specs=[
            pl.BlockSpec((RB, NC), lambda i: (i, 0)),
            pl.BlockSpec((NC, RB, DH), lambda i: (0, i, 0)),
            pl.BlockSpec((RB, DH), lambda i: (i, 0)),
            pl.BlockSpec((DOUT,), lambda i: (0,)),
            pl.BlockSpec((1, 1, RB), lambda i: (i, 0, 0)),
        ],
        out_specs=[
            pl.BlockSpec((RB, DOUT), lambda i: (i, 0)),
            pl.BlockSpec((G, DOUT), lambda i: (0, 0)),
        ],
        out_shape=[
            jax.ShapeDtypeStruct((N, DOUT), jnp.float32),
            jax.ShapeDtypeStruct((G, DOUT), jnp.float32),
        ],
    )(degt, s2p, hw2p, b2, batch3)


def kernel(x, edge_index, batch, W1, b1, W2, b2):
    src = edge_index[1]
    dst = edge_index[0]
    pad = jnp.full((EP - E,), N, dtype=jnp.int32)
    srcw = jnp.concatenate([src, pad]).reshape(NW, KB, LB)
    dstw = jnp.concatenate([dst, pad]).reshape(NW, KB, LB)

    z1 = jnp.zeros((NP,), jnp.float32)
    z128 = jnp.zeros((NP, DH), jnp.float32)
    ones = jnp.ones((LB,), jnp.float32)

    degp = _sc_deg(dstw, z1, ones).reshape(NC, NP)
    degt = degp.T
    hw1p = _tc_lin1(x, W1, degt)
    s1p = _sc_scatter(hw1p, srcw, dstw, z128, DH)
    hw2p = _tc_lin2(s1p, hw1p, degt, b1, W2)
    s2p = _sc_scatter(hw2p, srcw, dstw, z128, DH)
    batch3 = batch.reshape(NB, 1, RB)
    h2, hs = _tc_out(s2p, hw2p, degt, b2, batch3)
    return (hs, h2)
